# Initial kernel scaffold; baseline (speedup 1.0000x reference)
import jax
import jax.numpy as jnp
from jax.experimental import pallas as pl


def kernel(batched_events, mask, emb_w, emb_b, lstm_w, lstm_b, d1_w, d1_b, d2_w, d2_b, d3_w, d3_b):
    raise NotImplementedError("write your pallas kernel here")


# speedup vs baseline: 5.3898x; 5.3898x over previous
import functools

import jax
import jax.numpy as jnp
from jax import lax
from jax.experimental import pallas as pl
from jax.experimental.pallas import tpu as pltpu
from jax.experimental.pallas import tpu_sc as plsc

H = 128
W = 128
ED = 128
HD = 64
B = 4
N = 65536
HW = H * W
TOT = B * N

_NC = 2
_NS = 16
_EV_PER_TILE = TOT // (_NC * _NS)
_CHUNK = 2048
_NCH = _EV_PER_TILE // _CHUNK
_SUB = 128
_NSUB = _CHUNK // _SUB


def _sc_splat(ev8, xcol, ycol, zrows):
    mesh = plsc.VectorSubcoreMesh(core_axis_name="c", subcore_axis_name="s")

    @functools.partial(
        pl.kernel,
        out_type=jax.ShapeDtypeStruct((B * HW, 8), jnp.float32),
        mesh=mesh,
        scratch_types=[
            pltpu.VMEM_SHARED((2 * HW, 8), jnp.float32),
            pltpu.VMEM((_CHUNK, 8), jnp.float32),
            pltpu.VMEM((_CHUNK,), jnp.float32),
            pltpu.VMEM((_CHUNK,), jnp.float32),
            pltpu.VMEM((_NSUB, _SUB), jnp.int32),
            pltpu.SemaphoreType.DMA,
        ],
        compiler_params=pltpu.CompilerParams(use_tc_tiling_on_sc=False),
    )
    def k(ev_hbm, x_hbm, y_hbm, z_hbm, out_hbm, acc, evb, xb, yb, idxb, sem):
        c = lax.axis_index("c")
        s = lax.axis_index("s")
        wid = c * _NS + s
        local_b = s // 8

        pltpu.sync_copy(z_hbm, acc.at[pl.ds(s * _CHUNK, _CHUNK)])
        plsc.subcore_barrier()

        def chunk_body(t, carry):
            base = wid * _EV_PER_TILE + t * _CHUNK
            pltpu.sync_copy(ev_hbm.at[pl.ds(base, _CHUNK)], evb)
            pltpu.sync_copy(x_hbm.at[pl.ds(base, _CHUNK)], xb)
            pltpu.sync_copy(y_hbm.at[pl.ds(base, _CHUNK)], yb)
            for r in range(_NSUB):
                for j in range(_SUB // 16):
                    o = r * _SUB + j * 16
                    xv = xb[pl.ds(o, 16)]
                    yv = yb[pl.ds(o, 16)]
                    xi = jnp.clip((xv * W).astype(jnp.int32), 0, W - 1)
                    yi = jnp.clip((yv * H).astype(jnp.int32), 0, H - 1)
                    idxb[r, pl.ds(j * 16, 16)] = local_b * HW + yi * W + xi
            descs = []
            for r in range(_NSUB):
                descs.append(pltpu.async_copy(
                    evb.at[pl.ds(r * _SUB, _SUB)],
                    acc.at[idxb.at[r]],
                    sem, add=True))
            for d in descs:
                d.wait()
            return carry

        lax.fori_loop(0, _NCH, chunk_body, 0)
        plsc.subcore_barrier()
        pltpu.sync_copy(acc.at[pl.ds(s * _CHUNK, _CHUNK)],
                        out_hbm.at[pl.ds(wid * _CHUNK, _CHUNK)])

    return k(ev8, xcol, ycol, zrows)


_RC = 2048
_NRC = HW // _RC
_PAD = 136
_OFFS = [(dy, dx) for dy in (-1, 0, 1) for dx in (-1, 0, 1)]


def _col_masks(dtype):
    xpos = lax.broadcasted_iota(jnp.int32, (HW, 1), 0) % W
    mneg = (xpos != 0).astype(dtype)
    mpos = (xpos != W - 1).astype(dtype)
    return mneg, mpos


def _im2col(padded, r0, mneg, mpos, nch):
    cols = []
    for (dy, dx) in _OFFS:
        sft = dy * W + dx
        win = lax.slice_in_dim(padded, _PAD + r0 + sft, _PAD + r0 + sft + _RC, axis=0)
        if dx == -1:
            win = win * lax.slice_in_dim(mneg, r0, r0 + _RC, axis=0)
        elif dx == 1:
            win = win * lax.slice_in_dim(mpos, r0, r0 + _RC, axis=0)
        cols.append(win)
    return jnp.concatenate(cols, axis=1)


def _gates_body(acc_ref, w8_ref, wg_ref, bg_ref, h_ref):
    a = acc_ref[0]
    cnt = a[:, 4:5]
    grid = jnp.dot(a, w8_ref[...], preferred_element_type=jnp.float32)
    grid = (grid * (1.0 / (cnt + 1e-6))).astype(jnp.bfloat16)
    gp = jnp.pad(grid, ((_PAD, _PAD), (0, 0)))
    mneg, mpos = _col_masks(jnp.bfloat16)
    bg = bg_ref[...].astype(jnp.float32)
    for rc in range(_NRC):
        r0 = rc * _RC
        x = _im2col(gp, r0, mneg, mpos, ED)
        g = jnp.dot(x, wg_ref[...], preferred_element_type=jnp.float32) + bg
        gi = jax.nn.sigmoid(g[:, :HD])
        go = jax.nn.sigmoid(g[:, HD:2 * HD])
        gg = jnp.tanh(g[:, 2 * HD:])
        h = go * jnp.tanh(gi * gg)
        h_ref[0, r0:r0 + _RC, :] = h.astype(jnp.bfloat16)


def _decoder_body(h_ref, w1_ref, b1_ref, w2_ref, b2_ref, w3_ref, b3_ref,
                  out_ref, x1p_ref):
    mneg, mpos = _col_masks(jnp.bfloat16)
    hh = h_ref[0]
    hp = jnp.pad(hh, ((_PAD, _PAD), (0, 0)))
    b1 = b1_ref[...].astype(jnp.float32)
    x1p_ref[:_PAD, :] = jnp.zeros((_PAD, HD), jnp.bfloat16)
    x1p_ref[_PAD + HW:, :] = jnp.zeros((_PAD, HD), jnp.bfloat16)
    for rc in range(_NRC):
        r0 = rc * _RC
        x = _im2col(hp, r0, mneg, mpos, HD)
        x1 = jnp.dot(x, w1_ref[...], preferred_element_type=jnp.float32) + b1
        x1p_ref[_PAD + r0:_PAD + r0 + _RC, :] = jnp.maximum(x1, 0.0).astype(jnp.bfloat16)
    b2 = b2_ref[...].astype(jnp.float32)
    w3 = w3_ref[...].astype(jnp.float32)
    b3 = b3_ref[0, 0].astype(jnp.float32)
    x1p = x1p_ref[...]
    for rc in range(_NRC):
        r0 = rc * _RC
        x = _im2col(x1p, r0, mneg, mpos, HD)
        x2 = jnp.dot(x, w2_ref[...], preferred_element_type=jnp.float32) + b2
        x2 = jnp.maximum(x2, 0.0)
        dvec = jnp.sum(x2 * w3, axis=1) + b3
        dvec = jax.nn.sigmoid(dvec)
        out_ref[0, rc * (_RC // W):(rc + 1) * (_RC // W), :] = (
            dvec.reshape(_RC // W, W))


def kernel(batched_events, mask, emb_w, emb_b, lstm_w, lstm_b,
           d1_w, d1_b, d2_w, d2_b, d3_w, d3_b):
    f32 = jnp.float32
    ev = batched_events.reshape(TOT, 4).astype(f32)
    m = mask.reshape(TOT, 1).astype(f32)
    ev8 = jnp.concatenate([ev * m, m, jnp.zeros((TOT, 3), f32)], axis=1)
    xcol = ev[:, 1]
    ycol = ev[:, 2]
    zrows = jnp.zeros((_CHUNK, 8), f32)

    acc = _sc_splat(ev8, xcol, ycol, zrows)
    acc = acc.reshape(B, HW, 8)

    w8 = jnp.concatenate([emb_w, emb_b[None, :], jnp.zeros((3, ED), f32)], 0)

    sel = jnp.concatenate([jnp.arange(0, HD), jnp.arange(2 * HD, 3 * HD),
                           jnp.arange(3 * HD, 4 * HD)])
    wg = lstm_w[sel][:, :ED]
    wg = wg.transpose(2, 3, 1, 0).reshape(9 * ED, 3 * HD).astype(jnp.bfloat16)
    bg = lstm_b[sel][None, :]

    h = pl.pallas_call(
        _gates_body,
        grid=(B,),
        in_specs=[
            pl.BlockSpec((1, HW, 8), lambda b: (b, 0, 0)),
            pl.BlockSpec((8, ED), lambda b: (0, 0)),
            pl.BlockSpec((9 * ED, 3 * HD), lambda b: (0, 0)),
            pl.BlockSpec((1, 3 * HD), lambda b: (0, 0)),
        ],
        out_specs=pl.BlockSpec((1, HW, HD), lambda b: (b, 0, 0)),
        out_shape=jax.ShapeDtypeStruct((B, HW, HD), jnp.bfloat16),
    )(acc, w8, wg, bg)

    w1 = d1_w[:, :, :, :].transpose(2, 3, 1, 0).reshape(9 * HD, HD)
    w1 = w1.astype(jnp.bfloat16)
    b1 = d1_b[None, :]
    w2 = d2_w.transpose(2, 3, 1, 0).reshape(9 * HD, 32).astype(jnp.bfloat16)
    b2 = d2_b[None, :]
    w3 = d3_w.reshape(1, 32)
    b3 = d3_b.reshape(1, 1)

    depth = pl.pallas_call(
        _decoder_body,
        grid=(B,),
        in_specs=[
            pl.BlockSpec((1, HW, HD), lambda b: (b, 0, 0)),
            pl.BlockSpec((9 * HD, HD), lambda b: (0, 0)),
            pl.BlockSpec((1, HD), lambda b: (0, 0)),
            pl.BlockSpec((9 * HD, 32), lambda b: (0, 0)),
            pl.BlockSpec((1, 32), lambda b: (0, 0)),
            pl.BlockSpec((1, 32), lambda b: (0, 0)),
            pl.BlockSpec((1, 1), lambda b: (0, 0)),
        ],
        out_specs=pl.BlockSpec((1, H, W), lambda b: (b, 0, 0)),
        out_shape=jax.ShapeDtypeStruct((B, H, W), f32),
        scratch_shapes=[pltpu.VMEM((HW + 2 * _PAD, HD), jnp.bfloat16)],
    )(h, w1, b1, w2, b2, w3, b3)
    return depth
